# baseline (device time: 90769 ns/iter reference)
import jax
import jax.numpy as jnp
from jax import lax
from jax.experimental import pallas as pl
from jax.experimental.pallas import tpu as pltpu

import os

N_DEV = 8
NT = 512
_SKIP_A2A = os.environ.get("KERNEL_SKIP_A2A", "0") == "1"
_SKIP_GEMM = os.environ.get("KERNEL_SKIP_GEMM", "0") == "1"
_A2A_ONLY = os.environ.get("KERNEL_A2A_ONLY", "0") == "1"
_BARRIER_ONLY = os.environ.get("KERNEL_BARRIER_ONLY", "0") == "1"


def kernel(x, w_mat, scale_x, scale_w):
    k_full, k_loc = x.shape
    _, n = w_mat.shape
    m_loc = k_full // N_DEV
    n_tiles = n // NT

    def body(x_ref, w_hbm, sx_ref, sw_ref, out_ref,
             xf8_ref, xg_ref, wtile_ref, wbf_ref, copy_sems, send_sems,
             recv_sems):
        i = lax.axis_index("i")

        barrier_sem = pltpu.get_barrier_semaphore()
        for s in range(1, N_DEV):
            pl.semaphore_signal(
                barrier_sem, inc=1,
                device_id=(lax.rem(i + s, N_DEV),),
                device_id_type=pl.DeviceIdType.MESH,
            )
        pl.semaphore_wait(barrier_sem, N_DEV - 1)

        for t in range(2) if not _A2A_ONLY else []:
            pltpu.make_async_copy(
                w_hbm.at[:, pl.ds(t * NT, NT)], wtile_ref.at[t], copy_sems.at[t]
            ).start()

        xf8_ref[:, :] = x_ref[:, :].astype(jnp.float8_e4m3fn)

        xg_ref[:, pl.ds(i * k_loc, k_loc)] = xf8_ref[pl.ds(i * m_loc, m_loc), :]

        sends = []
        for s in range(1, N_DEV) if not (_SKIP_A2A or _BARRIER_ONLY) else []:
            dst = lax.rem(i + s, N_DEV)
            rdma = pltpu.make_async_remote_copy(
                src_ref=xf8_ref.at[pl.ds(dst * m_loc, m_loc), :],
                dst_ref=xg_ref.at[:, pl.ds(i * k_loc, k_loc)],
                send_sem=send_sems.at[s - 1],
                recv_sem=recv_sems.at[s - 1],
                device_id=(dst,),
                device_id_type=pl.DeviceIdType.MESH,
            )
            rdma.start()
            sends.append(rdma)

        if not _A2A_ONLY:
            pltpu.make_async_copy(
                w_hbm.at[:, pl.ds(0, NT)], wtile_ref.at[0], copy_sems.at[0]
            ).wait()
            wbf_ref[0, :, :] = wtile_ref[0, :, :].astype(jnp.bfloat16)
            pltpu.make_async_copy(
                w_hbm.at[:, pl.ds(2 * NT, NT)], wtile_ref.at[0], copy_sems.at[0]
            ).start()

        for s in range(1, N_DEV) if not (_SKIP_A2A or _BARRIER_ONLY) else []:
            src = lax.rem(i - s + N_DEV, N_DEV)
            recv = pltpu.make_async_remote_copy(
                src_ref=xf8_ref.at[pl.ds(0, m_loc), :],
                dst_ref=xg_ref.at[:, pl.ds(src * k_loc, k_loc)],
                send_sem=send_sems.at[s - 1],
                recv_sem=recv_sems.at[s - 1],
                device_id=(i,),
                device_id_type=pl.DeviceIdType.MESH,
            )
            recv.wait_recv()

        xg_bf = xg_ref[:, :].astype(jnp.bfloat16)
        scale = sx_ref[0] * sw_ref[0]

        def gemm_step(t, carry):
            nxt = lax.rem(t + 1, 2)

            @pl.when(t + 1 < n_tiles)
            def _():
                pltpu.make_async_copy(
                    w_hbm.at[:, pl.ds((t + 1) * NT, NT)], wtile_ref.at[nxt],
                    copy_sems.at[nxt],
                ).wait()
                wbf_ref[nxt, :, :] = wtile_ref[nxt, :, :].astype(jnp.bfloat16)

                @pl.when(t + 3 < n_tiles)
                def _():
                    pltpu.make_async_copy(
                        w_hbm.at[:, pl.ds((t + 3) * NT, NT)],
                        wtile_ref.at[nxt], copy_sems.at[nxt],
                    ).start()

            acc = lax.dot_general(
                xg_bf, wbf_ref[lax.rem(t, 2), :, :], (((1,), (0,)), ((), ())),
                preferred_element_type=jnp.float32,
            )
            y = acc * scale
            out_ref[:, pl.ds(t * NT, NT)] = y * jax.nn.sigmoid(y)
            return carry

        def dma_only_step(t, carry):
            slot = lax.rem(t, 2)
            pltpu.make_async_copy(
                w_hbm.at[:, pl.ds(t * NT, NT)], wtile_ref.at[slot],
                copy_sems.at[slot],
            ).wait()
            out_ref[:, pl.ds(t * NT, NT)] = jnp.zeros((m_loc, NT), jnp.float32)

            @pl.when(t + 2 < n_tiles)
            def _():
                pltpu.make_async_copy(
                    w_hbm.at[:, pl.ds((t + 2) * NT, NT)], wtile_ref.at[slot],
                    copy_sems.at[slot],
                ).start()

            return carry

        if _A2A_ONLY:
            out_ref[:, :] = jnp.zeros((m_loc, n), jnp.float32)
            out_ref[:, pl.ds(0, k_full)] = xg_ref[:, :].astype(jnp.float32)
        else:
            lax.fori_loop(0, n_tiles, dma_only_step if _SKIP_GEMM else gemm_step, 0)

        for rdma in sends:
            rdma.wait_send()

    return pl.pallas_call(
        body,
        out_shape=jax.ShapeDtypeStruct((m_loc, n), jnp.float32),
        in_specs=[
            pl.BlockSpec(memory_space=pltpu.VMEM),
            pl.BlockSpec(memory_space=pl.ANY),
            pl.BlockSpec(memory_space=pltpu.SMEM),
            pl.BlockSpec(memory_space=pltpu.SMEM),
        ],
        out_specs=pl.BlockSpec(memory_space=pltpu.VMEM),
        scratch_shapes=[
            pltpu.VMEM((k_full, k_loc), jnp.float8_e4m3fn),
            pltpu.VMEM((m_loc, k_full), jnp.float8_e4m3fn),
            pltpu.VMEM((2, k_full, NT), jnp.float32),
            pltpu.VMEM((2, k_full, NT), jnp.bfloat16),
            pltpu.SemaphoreType.DMA((2,)),
            pltpu.SemaphoreType.DMA((N_DEV - 1,)),
            pltpu.SemaphoreType.DMA((N_DEV - 1,)),
        ],
        compiler_params=pltpu.CompilerParams(
            vmem_limit_bytes=64 * 1024 * 1024,
            collective_id=0,
        ),
    )(x, w_mat, scale_x, scale_w)


# device time: 78675 ns/iter; 1.1537x vs baseline; 1.1537x over previous
import jax
import jax.numpy as jnp
from jax import lax
from jax.experimental import pallas as pl
from jax.experimental.pallas import tpu as pltpu

import os

N_DEV = 8
NT = 512
_SKIP_A2A = os.environ.get("KERNEL_SKIP_A2A", "0") == "1"
_SKIP_GEMM = os.environ.get("KERNEL_SKIP_GEMM", "0") == "1"
_A2A_ONLY = os.environ.get("KERNEL_A2A_ONLY", "0") == "1"
_BARRIER_ONLY = os.environ.get("KERNEL_BARRIER_ONLY", "0") == "1"


def kernel(x, w_mat, scale_x, scale_w):
    k_full, k_loc = x.shape
    _, n = w_mat.shape
    m_loc = k_full // N_DEV
    n_tiles = n // NT

    def body(x_ref, w_hbm, sx_ref, sw_ref, out_ref,
             xf8_ref, xg_ref, wtile_ref, wbf_ref, copy_sems, send_sems,
             recv_sems):
        i = lax.axis_index("i")

        barrier_sem = pltpu.get_barrier_semaphore()
        for s in range(1, N_DEV):
            pl.semaphore_signal(
                barrier_sem, inc=1,
                device_id=(lax.rem(i + s, N_DEV),),
                device_id_type=pl.DeviceIdType.MESH,
            )
        pl.semaphore_wait(barrier_sem, N_DEV - 1)

        for t in range(2) if not _A2A_ONLY else []:
            pltpu.make_async_copy(
                w_hbm.at[:, pl.ds(t * NT, NT)], wtile_ref.at[t], copy_sems.at[t]
            ).start()

        xf8_ref[:, :] = x_ref[:, :].astype(jnp.float8_e4m3fn)

        xg_ref[:, pl.ds(i * k_loc, k_loc)] = xf8_ref[pl.ds(i * m_loc, m_loc), :]

        sends = []
        for s in range(1, N_DEV) if not (_SKIP_A2A or _BARRIER_ONLY) else []:
            dst = lax.rem(i + s, N_DEV)
            rdma = pltpu.make_async_remote_copy(
                src_ref=xf8_ref.at[pl.ds(dst * m_loc, m_loc), :],
                dst_ref=xg_ref.at[:, pl.ds(i * k_loc, k_loc)],
                send_sem=send_sems.at[s - 1],
                recv_sem=recv_sems.at[s - 1],
                device_id=(dst,),
                device_id_type=pl.DeviceIdType.MESH,
            )
            rdma.start()
            sends.append(rdma)

        if not _A2A_ONLY:
            pltpu.make_async_copy(
                w_hbm.at[:, pl.ds(0, NT)], wtile_ref.at[0], copy_sems.at[0]
            ).wait()
            wbf_ref[0, :, :] = wtile_ref[0, :, :].astype(jnp.float8_e4m3fn)
            pltpu.make_async_copy(
                w_hbm.at[:, pl.ds(2 * NT, NT)], wtile_ref.at[0], copy_sems.at[0]
            ).start()

        for s in range(1, N_DEV) if not (_SKIP_A2A or _BARRIER_ONLY) else []:
            src = lax.rem(i - s + N_DEV, N_DEV)
            recv = pltpu.make_async_remote_copy(
                src_ref=xf8_ref.at[pl.ds(0, m_loc), :],
                dst_ref=xg_ref.at[:, pl.ds(src * k_loc, k_loc)],
                send_sem=send_sems.at[s - 1],
                recv_sem=recv_sems.at[s - 1],
                device_id=(i,),
                device_id_type=pl.DeviceIdType.MESH,
            )
            recv.wait_recv()

        scale = sx_ref[0] * sw_ref[0]

        def gemm_step(t, carry):
            nxt = lax.rem(t + 1, 2)

            @pl.when(t + 1 < n_tiles)
            def _():
                pltpu.make_async_copy(
                    w_hbm.at[:, pl.ds((t + 1) * NT, NT)], wtile_ref.at[nxt],
                    copy_sems.at[nxt],
                ).wait()
                wbf_ref[nxt, :, :] = wtile_ref[nxt, :, :].astype(jnp.float8_e4m3fn)

                @pl.when(t + 3 < n_tiles)
                def _():
                    pltpu.make_async_copy(
                        w_hbm.at[:, pl.ds((t + 3) * NT, NT)],
                        wtile_ref.at[nxt], copy_sems.at[nxt],
                    ).start()

            acc = lax.dot_general(
                xg_ref[:, :], wbf_ref[lax.rem(t, 2), :, :],
                (((1,), (0,)), ((), ())),
                preferred_element_type=jnp.float32,
            )
            y = acc * scale
            out_ref[:, pl.ds(t * NT, NT)] = y * jax.nn.sigmoid(y)
            return carry

        def dma_only_step(t, carry):
            slot = lax.rem(t, 2)
            pltpu.make_async_copy(
                w_hbm.at[:, pl.ds(t * NT, NT)], wtile_ref.at[slot],
                copy_sems.at[slot],
            ).wait()
            out_ref[:, pl.ds(t * NT, NT)] = jnp.zeros((m_loc, NT), jnp.float32)

            @pl.when(t + 2 < n_tiles)
            def _():
                pltpu.make_async_copy(
                    w_hbm.at[:, pl.ds((t + 2) * NT, NT)], wtile_ref.at[slot],
                    copy_sems.at[slot],
                ).start()

            return carry

        if _A2A_ONLY:
            out_ref[:, :] = jnp.zeros((m_loc, n), jnp.float32)
            out_ref[:, pl.ds(0, k_full)] = xg_ref[:, :].astype(jnp.float32)
        else:
            lax.fori_loop(0, n_tiles, dma_only_step if _SKIP_GEMM else gemm_step, 0)

        for rdma in sends:
            rdma.wait_send()

    return pl.pallas_call(
        body,
        out_shape=jax.ShapeDtypeStruct((m_loc, n), jnp.float32),
        in_specs=[
            pl.BlockSpec(memory_space=pltpu.VMEM),
            pl.BlockSpec(memory_space=pl.ANY),
            pl.BlockSpec(memory_space=pltpu.SMEM),
            pl.BlockSpec(memory_space=pltpu.SMEM),
        ],
        out_specs=pl.BlockSpec(memory_space=pltpu.VMEM),
        scratch_shapes=[
            pltpu.VMEM((k_full, k_loc), jnp.float8_e4m3fn),
            pltpu.VMEM((m_loc, k_full), jnp.float8_e4m3fn),
            pltpu.VMEM((2, k_full, NT), jnp.float32),
            pltpu.VMEM((2, k_full, NT), jnp.float8_e4m3fn),
            pltpu.SemaphoreType.DMA((2,)),
            pltpu.SemaphoreType.DMA((N_DEV - 1,)),
            pltpu.SemaphoreType.DMA((N_DEV - 1,)),
        ],
        compiler_params=pltpu.CompilerParams(
            vmem_limit_bytes=64 * 1024 * 1024,
            collective_id=0,
        ),
    )(x, w_mat, scale_x, scale_w)


# device time: 77809 ns/iter; 1.1666x vs baseline; 1.0111x over previous
import os

import jax
import jax.numpy as jnp
from jax import lax
from jax.experimental import pallas as pl
from jax.experimental.pallas import tpu as pltpu

N_DEV = 8
NT = 512
RING = 8
_SKIP_A2A = os.environ.get("KERNEL_SKIP_A2A", "0") == "1"


def kernel(x, w_mat, scale_x, scale_w):
    k_full, k_loc = x.shape
    _, n = w_mat.shape
    m_loc = k_full // N_DEV
    n_tiles = n // NT

    def body(x_ref, w_hbm, sx_ref, sw_ref, out_hbm,
             xf8_ref, xg_ref, wtile_ref, wf8_ref, otile_ref,
             copy_sems, out_sems, send_sems, recv_sems):
        i = lax.axis_index("i")

        barrier_sem = pltpu.get_barrier_semaphore()
        for s in range(1, N_DEV):
            pl.semaphore_signal(
                barrier_sem, inc=1,
                device_id=(lax.rem(i + s, N_DEV),),
                device_id_type=pl.DeviceIdType.MESH,
            )
        pl.semaphore_wait(barrier_sem, N_DEV - 1)

        for t in range(2):
            pltpu.make_async_copy(
                w_hbm.at[:, pl.ds(t * NT, NT)], wtile_ref.at[t], copy_sems.at[t]
            ).start()

        xf8_ref[:, :] = x_ref[:, :].astype(jnp.float8_e4m3fn)

        xg_ref[:, pl.ds(i * k_loc, k_loc)] = xf8_ref[pl.ds(i * m_loc, m_loc), :]

        sends = []
        for s in range(1, N_DEV) if not _SKIP_A2A else []:
            dst = lax.rem(i + s, N_DEV)
            rdma = pltpu.make_async_remote_copy(
                src_ref=xf8_ref.at[pl.ds(dst * m_loc, m_loc), :],
                dst_ref=xg_ref.at[:, pl.ds(i * k_loc, k_loc)],
                send_sem=send_sems.at[s - 1],
                recv_sem=recv_sems.at[s - 1],
                device_id=(dst,),
                device_id_type=pl.DeviceIdType.MESH,
            )
            rdma.start()
            sends.append(rdma)

        for c in range(RING):
            fslot = c % 2
            pltpu.make_async_copy(
                w_hbm.at[:, pl.ds(c * NT, NT)], wtile_ref.at[fslot],
                copy_sems.at[fslot],
            ).wait()
            wf8_ref[c, :, :] = wtile_ref[fslot, :, :].astype(jnp.float8_e4m3fn)

            @pl.when(c + 2 < n_tiles)
            def _():
                pltpu.make_async_copy(
                    w_hbm.at[:, pl.ds((c + 2) * NT, NT)], wtile_ref.at[fslot],
                    copy_sems.at[fslot],
                ).start()

        for s in range(1, N_DEV) if not _SKIP_A2A else []:
            src = lax.rem(i - s + N_DEV, N_DEV)
            recv = pltpu.make_async_remote_copy(
                src_ref=xf8_ref.at[pl.ds(0, m_loc), :],
                dst_ref=xg_ref.at[:, pl.ds(src * k_loc, k_loc)],
                send_sem=send_sems.at[s - 1],
                recv_sem=recv_sems.at[s - 1],
                device_id=(i,),
                device_id_type=pl.DeviceIdType.MESH,
            )
            recv.wait_recv()

        scale = sx_ref[0] * sw_ref[0]

        def gemm_step(t, carry):
            oslot = lax.rem(t, 2)

            @pl.when(t >= 2)
            def _():
                pltpu.make_async_copy(
                    otile_ref.at[oslot],
                    out_hbm.at[:, pl.ds((t - 2) * NT, NT)],
                    out_sems.at[oslot],
                ).wait()

            acc = lax.dot_general(
                xg_ref[:, :], wf8_ref[lax.rem(t, RING), :, :],
                (((1,), (0,)), ((), ())),
                preferred_element_type=jnp.float32,
            )
            y = acc * scale
            otile_ref[oslot, :, :] = y * jax.nn.sigmoid(y)
            pltpu.make_async_copy(
                otile_ref.at[oslot], out_hbm.at[:, pl.ds(t * NT, NT)],
                out_sems.at[oslot],
            ).start()

            @pl.when(t + RING < n_tiles)
            def _():
                fslot = lax.rem(t, 2)
                pltpu.make_async_copy(
                    w_hbm.at[:, pl.ds((t + RING) * NT, NT)], wtile_ref.at[fslot],
                    copy_sems.at[fslot],
                ).wait()
                wf8_ref[lax.rem(t, RING), :, :] = (
                    wtile_ref[fslot, :, :].astype(jnp.float8_e4m3fn))

                @pl.when(t + RING + 2 < n_tiles)
                def _():
                    pltpu.make_async_copy(
                        w_hbm.at[:, pl.ds((t + RING + 2) * NT, NT)],
                        wtile_ref.at[fslot], copy_sems.at[fslot],
                    ).start()

            return carry

        lax.fori_loop(0, n_tiles, gemm_step, 0)

        for t in (n_tiles - 2, n_tiles - 1):
            pltpu.make_async_copy(
                otile_ref.at[t % 2], out_hbm.at[:, pl.ds(t * NT, NT)],
                out_sems.at[t % 2],
            ).wait()

        for rdma in sends:
            rdma.wait_send()

    return pl.pallas_call(
        body,
        out_shape=jax.ShapeDtypeStruct((m_loc, n), jnp.float32),
        in_specs=[
            pl.BlockSpec(memory_space=pltpu.VMEM),
            pl.BlockSpec(memory_space=pl.ANY),
            pl.BlockSpec(memory_space=pltpu.SMEM),
            pl.BlockSpec(memory_space=pltpu.SMEM),
        ],
        out_specs=pl.BlockSpec(memory_space=pl.ANY),
        scratch_shapes=[
            pltpu.VMEM((k_full, k_loc), jnp.float8_e4m3fn),
            pltpu.VMEM((m_loc, k_full), jnp.float8_e4m3fn),
            pltpu.VMEM((2, k_full, NT), jnp.float32),
            pltpu.VMEM((RING, k_full, NT), jnp.float8_e4m3fn),
            pltpu.VMEM((2, m_loc, NT), jnp.float32),
            pltpu.SemaphoreType.DMA((2,)),
            pltpu.SemaphoreType.DMA((2,)),
            pltpu.SemaphoreType.DMA((N_DEV - 1,)),
            pltpu.SemaphoreType.DMA((N_DEV - 1,)),
        ],
        compiler_params=pltpu.CompilerParams(
            vmem_limit_bytes=64 * 1024 * 1024,
            collective_id=0,
        ),
    )(x, w_mat, scale_x, scale_w)
